# baseline (speedup 1.0000x reference)
import functools

import jax
import jax.numpy as jnp
from jax import lax
from jax.experimental import pallas as pl
from jax.experimental.pallas import tpu as pltpu
from jax.experimental.pallas import tpu_sc as plsc

NC = 2
NS = 16
LANES = 16


def _sc_mesh():
    return plsc.VectorSubcoreMesh(
        core_axis_name="c", subcore_axis_name="s", num_cores=NC, num_subcores=NS
    )


@functools.lru_cache(maxsize=None)
def _make_deg_kernel(n, e):
    nw = NC * NS
    assert e % nw == 0 and n % LANES == 0
    epw = e // nw
    blk = 2000
    assert epw % blk == 0 and blk % LANES == 0

    @functools.partial(
        pl.kernel,
        out_type=jax.ShapeDtypeStruct((n // 1000, nw, 1000), jnp.float32),
        mesh=_sc_mesh(),
        scratch_types=[
            pltpu.VMEM((n,), jnp.float32),
            pltpu.VMEM((blk,), jnp.int32),
        ],
        compiler_params=pltpu.CompilerParams(needs_layout_passes=False, use_tc_tiling_on_sc=False),
    )
    def deg_kernel(dst_hbm, out_hbm, hist, dbuf):
        s = lax.axis_index("s")
        c = lax.axis_index("c")
        w = s * NC + c
        zero = jnp.zeros((LANES,), jnp.float32)

        def zbody(i, carry):
            hist[pl.ds(i * LANES, LANES)] = zero
            return carry

        lax.fori_loop(0, n // LANES, zbody, 0)

        ones = jnp.ones((LANES,), jnp.float32)
        ebase = w * epw

        def blk_body(bi, carry):
            pltpu.sync_copy(
                dst_hbm.at[pl.ds(ebase + bi * blk, blk)], dbuf
            )

            def body(i, c2):
                idx = dbuf[pl.ds(i * LANES, LANES)]
                plsc.addupdate_scatter(hist, [idx], ones)
                return c2

            lax.fori_loop(0, blk // LANES, body, 0)
            return carry

        lax.fori_loop(0, epw // blk, blk_body, 0)

        def out_body(b, carry):
            pltpu.sync_copy(hist.at[pl.ds(b * 1000, 1000)], out_hbm.at[b, w])
            return carry

        lax.fori_loop(0, n // 1000, out_body, 0)

    return deg_kernel


@functools.lru_cache(maxsize=None)
def _make_agg_kernel(n, e, f, c_rows, n_pass):
    assert e % NS == 0 and c_rows % (NS * LANES) == 0 and f % LANES == 0
    epw = e // NS
    blk = 2000
    assert epw % blk == 0
    cap = blk + 192
    nb = 128 if f <= 64 else 64
    c_tot = c_rows + 64
    n_out = n_pass * NC * c_rows
    assert n_out >= n

    @functools.partial(
        pl.kernel,
        out_type=jax.ShapeDtypeStruct((n_out, f), jnp.float32),
        mesh=_sc_mesh(),
        scratch_types=[
            pltpu.VMEM_SHARED((c_tot, f), jnp.float32),
            pltpu.VMEM((blk,), jnp.int32),
            pltpu.VMEM((blk,), jnp.int32),
            pltpu.VMEM((cap,), jnp.int32),
            pltpu.VMEM((cap,), jnp.int32),
            pltpu.VMEM((nb,), jnp.int32),
            pltpu.VMEM((nb,), jnp.int32),
            pltpu.VMEM((nb, f), jnp.float32),
            pltpu.VMEM((32, f), jnp.float32),
            pltpu.SemaphoreType.DMA,
        ],
        compiler_params=pltpu.CompilerParams(needs_layout_passes=False, use_tc_tiling_on_sc=False),
    )
    def agg_kernel(u_hbm, src_hbm, dst_hbm, out_hbm, acc, sbuf, dbuf, fsrc, fdst,
                   gstage, sstage, rows, zbuf, sem):
        s = lax.axis_index("s")
        c = lax.axis_index("c")
        w = s * NC + c
        zv = jnp.zeros((LANES,), jnp.float32)

        def zrow(i, carry):
            for j in range(f // LANES):
                zbuf[i, pl.ds(j * LANES, LANES)] = zv
            return carry

        lax.fori_loop(0, 32, zrow, 0)

        iota = lax.iota(jnp.int32, LANES)
        padsrc = (iota * 131 + w * 977) % n
        paddst = c_rows + iota

        rpt_tot = c_tot // NS
        rpt = c_rows // NS

        def pass_body(p, carry):
            chunk = p * NC + c
            base = chunk * c_rows

            nz = rpt_tot // 32

            def zb_body(i, c2):
                pltpu.sync_copy(zbuf, acc.at[pl.ds(s * rpt_tot + i * 32, 32)])
                return c2

            lax.fori_loop(0, nz, zb_body, 0)
            ztail = rpt_tot - nz * 32
            if ztail:
                pltpu.sync_copy(zbuf.at[pl.ds(0, ztail)],
                                acc.at[pl.ds(s * rpt_tot + nz * 32, ztail)])
            plsc.subcore_barrier()

            ebase = s * epw

            def blk_body(bi, cur):
                off = ebase + bi * blk
                pltpu.sync_copy(src_hbm.at[pl.ds(off, blk)], sbuf)
                pltpu.sync_copy(dst_hbm.at[pl.ds(off, blk)], dbuf)

                def cbody(i, cur):
                    srcv = sbuf[pl.ds(i * LANES, LANES)]
                    dstv = dbuf[pl.ds(i * LANES, LANES)]
                    m = (dstv >= base) & (dstv < base + c_rows)
                    plsc.store_compressed(fsrc.at[pl.ds(cur, LANES)], srcv,
                                          mask=m)
                    plsc.store_compressed(fdst.at[pl.ds(cur, LANES)],
                                          dstv - base, mask=m)
                    return cur + jnp.sum(m.astype(jnp.int32))

                cur = lax.fori_loop(0, blk // LANES, cbody, cur)

                nfull = cur // nb

                def fbody(k, c2):
                    kb = k * nb
                    for j in range(nb // LANES):
                        gstage[pl.ds(j * LANES, LANES)] = (
                            fsrc[pl.ds(kb + j * LANES, LANES)])
                        sstage[pl.ds(j * LANES, LANES)] = (
                            fdst[pl.ds(kb + j * LANES, LANES)])
                    pltpu.async_copy(u_hbm.at[gstage], rows, sem).wait()
                    pltpu.sync_copy(rows, acc.at[sstage], add=True)
                    return c2

                lax.fori_loop(0, nfull, fbody, 0)

                tb = nfull * nb
                for j in range(9):
                    fsrc[pl.ds(j * LANES, LANES)] = (
                        fsrc[pl.ds(tb + j * LANES, LANES)])
                    fdst[pl.ds(j * LANES, LANES)] = (
                        fdst[pl.ds(tb + j * LANES, LANES)])
                return cur - tb

            cur = lax.fori_loop(0, epw // blk, blk_body, jnp.int32(0))

            @pl.when(cur > 0)
            def _final():
                for j in range(nb // LANES):
                    fsrc[pl.ds(cur + j * LANES, LANES)] = padsrc
                    fdst[pl.ds(cur + j * LANES, LANES)] = paddst
                for j in range(nb // LANES):
                    gstage[pl.ds(j * LANES, LANES)] = (
                        fsrc[pl.ds(j * LANES, LANES)])
                    sstage[pl.ds(j * LANES, LANES)] = (
                        fdst[pl.ds(j * LANES, LANES)])
                pltpu.async_copy(u_hbm.at[gstage], rows, sem).wait()
                pltpu.sync_copy(rows, acc.at[sstage], add=True)

            plsc.subcore_barrier()

            obase = chunk * c_rows
            nf = rpt // nb

            def fl_body(i, c2):
                r = s * rpt + i * nb
                pltpu.sync_copy(acc.at[pl.ds(r, nb)], rows)
                pltpu.sync_copy(rows, out_hbm.at[pl.ds(obase + r, nb)])
                return c2

            lax.fori_loop(0, nf, fl_body, 0)
            ftail = rpt - nf * nb
            if ftail:
                r = s * rpt + nf * nb
                pltpu.sync_copy(acc.at[pl.ds(r, ftail)],
                                rows.at[pl.ds(0, ftail)])
                pltpu.sync_copy(rows.at[pl.ds(0, ftail)],
                                out_hbm.at[pl.ds(obase + r, ftail)])
            plsc.subcore_barrier()
            return carry

        lax.fori_loop(0, n_pass, pass_body, 0)

    return agg_kernel


_BR = 1000


@functools.lru_cache(maxsize=None)
def _make_dinv_kernel(n, nw):
    nblk = n // 1000

    def body(parts_ref, dinv_ref):
        deg = jnp.sum(parts_ref[0], axis=0) + 1.0
        dinv_ref[...] = jax.lax.rsqrt(deg)[None, None, :]

    return pl.pallas_call(
        body,
        grid=(nblk,),
        in_specs=[pl.BlockSpec((1, nw, 1000), lambda i: (i, 0, 0))],
        out_specs=pl.BlockSpec((1, 1, 1000), lambda i: (i, 0, 0)),
        out_shape=jax.ShapeDtypeStruct((nblk, 1, 1000), jnp.float32),
    )


@functools.lru_cache(maxsize=None)
def _make_t1_kernel(n, fin, fout):
    nblk = n // _BR

    def body(x_ref, dinv_ref, w_ref, u_ref):
        h = jnp.dot(x_ref[...], w_ref[...],
                    preferred_element_type=jnp.float32)
        u_ref[...] = h * dinv_ref[...]

    return pl.pallas_call(
        body,
        grid=(nblk,),
        in_specs=[
            pl.BlockSpec((_BR, fin), lambda i: (i, 0)),
            pl.BlockSpec((_BR, 1), lambda i: (i, 0)),
            pl.BlockSpec((fin, fout), lambda i: (0, 0)),
        ],
        out_specs=pl.BlockSpec((_BR, fout), lambda i: (i, 0)),
        out_shape=jax.ShapeDtypeStruct((n, fout), jnp.float32),
    )


@functools.lru_cache(maxsize=None)
def _make_t2_kernel(n, fin, fout):
    nblk = n // _BR

    def body(agg_ref, u_ref, dinv_ref, b_ref, w_ref, un_ref):
        dinv = dinv_ref[...]
        h = jax.nn.relu((agg_ref[...] + u_ref[...]) * dinv + b_ref[...])
        un_ref[...] = jnp.dot(h, w_ref[...],
                              preferred_element_type=jnp.float32) * dinv

    return pl.pallas_call(
        body,
        grid=(nblk,),
        in_specs=[
            pl.BlockSpec((_BR, fin), lambda i: (i, 0)),
            pl.BlockSpec((_BR, fin), lambda i: (i, 0)),
            pl.BlockSpec((_BR, 1), lambda i: (i, 0)),
            pl.BlockSpec((1, fin), lambda i: (0, 0)),
            pl.BlockSpec((fin, fout), lambda i: (0, 0)),
        ],
        out_specs=pl.BlockSpec((_BR, fout), lambda i: (i, 0)),
        out_shape=jax.ShapeDtypeStruct((n, fout), jnp.float32),
    )


@functools.lru_cache(maxsize=None)
def _make_head_kernel(n, f, g):
    nblk = n // _BR

    def body(agg_ref, u_ref, dinv_ref, b_ref, batch_ref, wf1_ref, bf1_ref,
             wf2_ref, bf2_ref, gamma_ref, beta_ref, out_ref, psum, cnt):
        i = pl.program_id(0)

        @pl.when(i == 0)
        def _init():
            psum[...] = jnp.zeros_like(psum)
            cnt[...] = jnp.zeros_like(cnt)

        h = jax.nn.relu((agg_ref[...] + u_ref[...]) * dinv_ref[...]
                        + b_ref[...])
        gid = batch_ref[...]
        onehot = (gid == lax.broadcasted_iota(jnp.int32, (1, g), 1)
                  ).astype(jnp.float32)
        psum[...] += jax.lax.dot_general(
            onehot, h, (((0,), (0,)), ((), ())),
            preferred_element_type=jnp.float32)
        cnt[...] += jnp.sum(onehot, axis=0, keepdims=True).T

        @pl.when(i == nblk - 1)
        def _fin():
            gm = psum[...] / jnp.maximum(cnt[...], 1.0)
            gm = jax.nn.relu(jnp.dot(gm, wf1_ref[...],
                                     preferred_element_type=jnp.float32)
                             + bf1_ref[...])
            gm = jax.nn.relu(jnp.dot(gm, wf2_ref[...],
                                     preferred_element_type=jnp.float32)
                             + bf2_ref[...])
            mu = jnp.mean(gm, axis=-1, keepdims=True)
            var = jnp.mean((gm - mu) ** 2, axis=-1, keepdims=True)
            out_ref[...] = ((gm - mu) / jnp.sqrt(var + 1e-5)
                            * gamma_ref[...] + beta_ref[...])

    return pl.pallas_call(
        body,
        grid=(nblk,),
        in_specs=[
            pl.BlockSpec((_BR, f), lambda i: (i, 0)),
            pl.BlockSpec((_BR, f), lambda i: (i, 0)),
            pl.BlockSpec((_BR, 1), lambda i: (i, 0)),
            pl.BlockSpec((1, f), lambda i: (0, 0)),
            pl.BlockSpec((_BR, 1), lambda i: (i, 0)),
            pl.BlockSpec((f, 128), lambda i: (0, 0)),
            pl.BlockSpec((1, 128), lambda i: (0, 0)),
            pl.BlockSpec((128, f), lambda i: (0, 0)),
            pl.BlockSpec((1, f), lambda i: (0, 0)),
            pl.BlockSpec((1, f), lambda i: (0, 0)),
            pl.BlockSpec((1, f), lambda i: (0, 0)),
        ],
        out_specs=pl.BlockSpec((g, f), lambda i: (0, 0)),
        out_shape=jax.ShapeDtypeStruct((g, f), jnp.float32),
        scratch_shapes=[
            pltpu.VMEM((g, f), jnp.float32),
            pltpu.VMEM((g, 1), jnp.float32),
        ],
    )


def kernel(x, edge_index, edge_attr, batch, W1, b1, W2, b2, W3, b3,
           Wf1, bf1, Wf2, bf2, gamma, beta):
    n = x.shape[0]
    e = edge_index.shape[1]
    num_graphs = 256

    e_src = edge_index[0]
    e_dst = edge_index[1]
    deg_parts = _make_deg_kernel(n, e)(e_dst)

    agg64 = _make_agg_kernel(n, e, 64, 25088, 2)
    agg128 = _make_agg_kernel(n, e, 128, 13568, 4)

    dinv = _make_dinv_kernel(n, NC * NS)(deg_parts).reshape(n, 1)
    u1 = _make_t1_kernel(n, 37, 64)(x, dinv, W1)
    agg1 = agg64(u1, e_src, e_dst)
    u2 = _make_t2_kernel(n, 64, 128)(
        agg1[:n], u1, dinv, b1.reshape(1, -1), W2)
    agg2 = agg128(u2, e_src, e_dst)
    u3 = _make_t2_kernel(n, 128, 64)(
        agg2[:n], u2, dinv, b2.reshape(1, -1), W3)
    agg3 = agg64(u3, e_src, e_dst)
    return _make_head_kernel(n, 64, num_graphs)(
        agg3[:n], u3, dinv, b3.reshape(1, -1), batch.reshape(-1, 1),
        Wf1, bf1.reshape(1, -1), Wf2, bf2.reshape(1, -1),
        gamma.reshape(1, -1), beta.reshape(1, -1))


# speedup vs baseline: 1.3630x; 1.3630x over previous
import functools

import jax
import jax.numpy as jnp
from jax import lax
from jax.experimental import pallas as pl
from jax.experimental.pallas import tpu as pltpu
from jax.experimental.pallas import tpu_sc as plsc

NC = 2
NS = 16
LANES = 16


def _sc_mesh():
    return plsc.VectorSubcoreMesh(
        core_axis_name="c", subcore_axis_name="s", num_cores=NC, num_subcores=NS
    )


@functools.lru_cache(maxsize=None)
def _make_deg_kernel(n, e):
    nw = NC * NS
    assert e % nw == 0 and n % LANES == 0 and n % 1000 == 0
    epw = e // nw
    blk = 2000
    assert epw % blk == 0 and blk % LANES == 0

    @functools.partial(
        pl.kernel,
        out_type=jax.ShapeDtypeStruct((n // 1000, nw, 1000), jnp.float32),
        mesh=_sc_mesh(),
        scratch_types=[
            pltpu.VMEM((n,), jnp.float32),
            pltpu.VMEM((blk,), jnp.int32),
        ],
        compiler_params=pltpu.CompilerParams(needs_layout_passes=False,
                                             use_tc_tiling_on_sc=False),
    )
    def deg_kernel(dst_hbm, out_hbm, hist, dbuf):
        s = lax.axis_index("s")
        c = lax.axis_index("c")
        w = s * NC + c
        zero = jnp.zeros((LANES,), jnp.float32)

        def zbody(i, carry):
            hist[pl.ds(i * LANES, LANES)] = zero
            return carry

        lax.fori_loop(0, n // LANES, zbody, 0)

        ones = jnp.ones((LANES,), jnp.float32)
        ebase = w * epw

        def blk_body(bi, carry):
            pltpu.sync_copy(
                dst_hbm.at[pl.ds(ebase + bi * blk, blk)], dbuf
            )

            def body(i, c2):
                idx = dbuf[pl.ds(i * LANES, LANES)]
                plsc.addupdate_scatter(hist, [idx], ones)
                return c2

            lax.fori_loop(0, blk // LANES, body, 0)
            return carry

        lax.fori_loop(0, epw // blk, blk_body, 0)

        def out_body(b, carry):
            pltpu.sync_copy(hist.at[pl.ds(b * 1000, 1000)], out_hbm.at[b, w])
            return carry

        lax.fori_loop(0, n // 1000, out_body, 0)

    return deg_kernel


@functools.lru_cache(maxsize=None)
def _make_agg_kernel(n, e, f, c_rows, n_pass):
    assert e % NS == 0 and c_rows % (NS * LANES) == 0 and f % LANES == 0
    epw = e // NS
    blk = 1000
    assert epw % (2 * blk) == 0
    npair = epw // (2 * blk)
    cap = 1312
    nb = 128 if f <= 64 else 64
    c_tot = c_rows + 32
    n_out = n_pass * NC * c_rows
    assert n_out >= n

    @functools.partial(
        pl.kernel,
        out_type=jax.ShapeDtypeStruct((n_out, f), jnp.float32),
        mesh=_sc_mesh(),
        scratch_types=[
            pltpu.VMEM_SHARED((c_tot, f), jnp.float32),
            pltpu.VMEM((2, blk), jnp.int32),
            pltpu.VMEM((2, blk), jnp.int32),
            pltpu.VMEM((cap,), jnp.int32),
            pltpu.VMEM((cap,), jnp.int32),
            pltpu.VMEM((2, nb), jnp.int32),
            pltpu.VMEM((2, nb, f), jnp.float32),
            pltpu.SemaphoreType.DMA,
            pltpu.SemaphoreType.DMA,
            pltpu.SemaphoreType.DMA,
            pltpu.SemaphoreType.DMA,
            pltpu.SemaphoreType.DMA,
            pltpu.SemaphoreType.DMA,
        ],
        compiler_params=pltpu.CompilerParams(needs_layout_passes=False,
                                             use_tc_tiling_on_sc=False),
    )
    def agg_kernel(u_hbm, src_hbm, dst_hbm, out_hbm, acc, sbuf, dbuf, fsrc,
                   fdst, sstage, rows, es0, es1, ed0, ed1, gs0, gs1):
        s = lax.axis_index("s")
        c = lax.axis_index("c")
        w = s * NC + c
        zv = jnp.zeros((LANES,), jnp.float32)
        gsem = (gs0, gs1)

        iota = lax.iota(jnp.int32, LANES)
        padsrc = (iota * 131 + w * 977) % n
        paddst = c_rows + iota

        def pf(i, carry):
            fsrc[pl.ds(i * LANES, LANES)] = padsrc
            fdst[pl.ds(i * LANES, LANES)] = paddst
            return carry

        lax.fori_loop(0, cap // LANES, pf, 0)

        def zero_rows_all():
            def zr(i, c2):
                sl = i // (nb * (f // LANES))
                rem = i % (nb * (f // LANES))
                r = rem // (f // LANES)
                col = (rem % (f // LANES)) * LANES
                rows[sl, r, pl.ds(col, LANES)] = zv
                return c2

            lax.fori_loop(0, 2 * nb * (f // LANES), zr, 0)

        rpt_tot = c_tot // NS
        rpt = c_rows // NS

        def pass_body(p, carry):
            chunk = p * NC + c
            base = chunk * c_rows

            zero_rows_all()

            nz = rpt_tot // nb

            def zb_body(i, c2):
                pltpu.sync_copy(rows.at[0],
                                acc.at[pl.ds(s * rpt_tot + i * nb, nb)])
                return c2

            lax.fori_loop(0, nz, zb_body, 0)
            ztail = rpt_tot - nz * nb
            if ztail:
                pltpu.sync_copy(rows.at[0, pl.ds(0, ztail)],
                                acc.at[pl.ds(s * rpt_tot + nz * nb, ztail)])
            plsc.subcore_barrier()

            ebase = s * epw

            def compact(slot, cur):
                def cbody(i, cur):
                    srcv = sbuf[slot, pl.ds(i * LANES, LANES)]
                    dstv = dbuf[slot, pl.ds(i * LANES, LANES)]
                    m = (dstv >= base) & (dstv < base + c_rows)
                    plsc.store_compressed(fsrc.at[pl.ds(cur, LANES)], srcv,
                                          mask=m)
                    plsc.store_compressed(fdst.at[pl.ds(cur, LANES)],
                                          dstv - base, mask=m)
                    return cur + jnp.sum(m.astype(jnp.int32))

                return lax.fori_loop(0, blk // LANES, cbody, cur)

            def fire_gather(slot, kb):
                for j in range(nb // LANES):
                    sstage[slot, pl.ds(j * LANES, LANES)] = (
                        fdst[pl.ds(kb + j * LANES, LANES)])
                pltpu.async_copy(u_hbm.at[fsrc.at[pl.ds(kb, nb)]],
                                 rows.at[slot], gsem[slot])

            def wait_gather(slot):
                pltpu.make_async_copy(u_hbm.at[pl.ds(0, nb)], rows.at[slot],
                                      gsem[slot]).wait()

            def flush_batches(cur):
                nfull = cur // nb

                @pl.when(nfull > 0)
                def _p():
                    fire_gather(0, 0)

                def g_body(j, c2):
                    k1 = 2 * j + 1

                    @pl.when(k1 < nfull)
                    def _f1():
                        fire_gather(1, k1 * nb)

                    wait_gather(0)
                    pltpu.sync_copy(rows.at[0], acc.at[sstage.at[0]],
                                    add=True)

                    @pl.when(k1 < nfull)
                    def _s1():
                        @pl.when(k1 + 1 < nfull)
                        def _f0():
                            fire_gather(0, (k1 + 1) * nb)

                        wait_gather(1)
                        pltpu.sync_copy(rows.at[1], acc.at[sstage.at[1]],
                                        add=True)

                    return c2

                lax.fori_loop(0, (nfull + 1) // 2, g_body, 0)

                tb = nfull * nb
                for j in range(9):
                    fsrc[pl.ds(j * LANES, LANES)] = (
                        fsrc[pl.ds(tb + j * LANES, LANES)])
                    fdst[pl.ds(j * LANES, LANES)] = (
                        fdst[pl.ds(tb + j * LANES, LANES)])
                return cur - tb

            pltpu.async_copy(src_hbm.at[pl.ds(ebase, blk)], sbuf.at[0], es0)
            pltpu.async_copy(dst_hbm.at[pl.ds(ebase, blk)], dbuf.at[0], ed0)

            def pair_body(i, cur):
                b0 = 2 * i
                off1 = ebase + (b0 + 1) * blk
                pltpu.async_copy(src_hbm.at[pl.ds(off1, blk)], sbuf.at[1],
                                 es1)
                pltpu.async_copy(dst_hbm.at[pl.ds(off1, blk)], dbuf.at[1],
                                 ed1)
                pltpu.make_async_copy(src_hbm.at[pl.ds(0, blk)], sbuf.at[0],
                                      es0).wait()
                pltpu.make_async_copy(dst_hbm.at[pl.ds(0, blk)], dbuf.at[0],
                                      ed0).wait()
                cur = compact(0, cur)
                cur = flush_batches(cur)

                @pl.when(i < npair - 1)
                def _f():
                    off2 = ebase + (b0 + 2) * blk
                    pltpu.async_copy(src_hbm.at[pl.ds(off2, blk)],
                                     sbuf.at[0], es0)
                    pltpu.async_copy(dst_hbm.at[pl.ds(off2, blk)],
                                     dbuf.at[0], ed0)

                pltpu.make_async_copy(src_hbm.at[pl.ds(0, blk)], sbuf.at[1],
                                      es1).wait()
                pltpu.make_async_copy(dst_hbm.at[pl.ds(0, blk)], dbuf.at[1],
                                      ed1).wait()
                cur = compact(1, cur)
                cur = flush_batches(cur)
                return cur

            cur = lax.fori_loop(0, npair, pair_body, jnp.int32(0))

            @pl.when(cur > 0)
            def _final():
                for j in range(nb // LANES):
                    m = (iota + j * LANES) < cur
                    dv = fdst[pl.ds(j * LANES, LANES)]
                    sstage[0, pl.ds(j * LANES, LANES)] = (
                        jnp.where(m, dv, paddst))
                pltpu.async_copy(u_hbm.at[fsrc.at[pl.ds(0, nb)]],
                                 rows.at[0], gs0)
                pltpu.make_async_copy(u_hbm.at[pl.ds(0, nb)], rows.at[0],
                                      gs0).wait()
                pltpu.sync_copy(rows.at[0], acc.at[sstage.at[0]], add=True)

            plsc.subcore_barrier()

            obase = chunk * c_rows
            nf = rpt // nb

            def fl_body(i, c2):
                r = s * rpt + i * nb
                pltpu.sync_copy(acc.at[pl.ds(r, nb)], rows.at[0])
                pltpu.sync_copy(rows.at[0], out_hbm.at[pl.ds(obase + r, nb)])
                return c2

            lax.fori_loop(0, nf, fl_body, 0)
            ftail = rpt - nf * nb
            if ftail:
                r = s * rpt + nf * nb
                pltpu.sync_copy(acc.at[pl.ds(r, ftail)],
                                rows.at[0, pl.ds(0, ftail)])
                pltpu.sync_copy(rows.at[0, pl.ds(0, ftail)],
                                out_hbm.at[pl.ds(obase + r, ftail)])
            plsc.subcore_barrier()
            return carry

        lax.fori_loop(0, n_pass, pass_body, 0)

    return agg_kernel


_BR = 1000


@functools.lru_cache(maxsize=None)
def _make_dinv_kernel(n, nw):
    nblk = n // 1000

    def body(parts_ref, dinv_ref):
        deg = jnp.sum(parts_ref[0], axis=0) + 1.0
        dinv_ref[...] = jax.lax.rsqrt(deg)[None, None, :]

    return pl.pallas_call(
        body,
        grid=(nblk,),
        in_specs=[pl.BlockSpec((1, nw, 1000), lambda i: (i, 0, 0))],
        out_specs=pl.BlockSpec((1, 1, 1000), lambda i: (i, 0, 0)),
        out_shape=jax.ShapeDtypeStruct((nblk, 1, 1000), jnp.float32),
    )


@functools.lru_cache(maxsize=None)
def _make_t1_kernel(n, fin, fout):
    nblk = n // _BR

    def body(x_ref, dinv_ref, w_ref, u_ref):
        h = jnp.dot(x_ref[...], w_ref[...],
                    preferred_element_type=jnp.float32)
        u_ref[...] = h * dinv_ref[...]

    return pl.pallas_call(
        body,
        grid=(nblk,),
        in_specs=[
            pl.BlockSpec((_BR, fin), lambda i: (i, 0)),
            pl.BlockSpec((_BR, 1), lambda i: (i, 0)),
            pl.BlockSpec((fin, fout), lambda i: (0, 0)),
        ],
        out_specs=pl.BlockSpec((_BR, fout), lambda i: (i, 0)),
        out_shape=jax.ShapeDtypeStruct((n, fout), jnp.float32),
    )


@functools.lru_cache(maxsize=None)
def _make_t2_kernel(n, fin, fout):
    nblk = n // _BR

    def body(agg_ref, u_ref, dinv_ref, b_ref, w_ref, un_ref):
        dinv = dinv_ref[...]
        h = jax.nn.relu((agg_ref[...] + u_ref[...]) * dinv + b_ref[...])
        un_ref[...] = jnp.dot(h, w_ref[...],
                              preferred_element_type=jnp.float32) * dinv

    return pl.pallas_call(
        body,
        grid=(nblk,),
        in_specs=[
            pl.BlockSpec((_BR, fin), lambda i: (i, 0)),
            pl.BlockSpec((_BR, fin), lambda i: (i, 0)),
            pl.BlockSpec((_BR, 1), lambda i: (i, 0)),
            pl.BlockSpec((1, fin), lambda i: (0, 0)),
            pl.BlockSpec((fin, fout), lambda i: (0, 0)),
        ],
        out_specs=pl.BlockSpec((_BR, fout), lambda i: (i, 0)),
        out_shape=jax.ShapeDtypeStruct((n, fout), jnp.float32),
    )


@functools.lru_cache(maxsize=None)
def _make_head_kernel(n, f, g):
    nblk = n // _BR

    def body(agg_ref, u_ref, dinv_ref, b_ref, batch_ref, wf1_ref, bf1_ref,
             wf2_ref, bf2_ref, gamma_ref, beta_ref, out_ref, psum, cnt):
        i = pl.program_id(0)

        @pl.when(i == 0)
        def _init():
            psum[...] = jnp.zeros_like(psum)
            cnt[...] = jnp.zeros_like(cnt)

        h = jax.nn.relu((agg_ref[...] + u_ref[...]) * dinv_ref[...]
                        + b_ref[...])
        gid = batch_ref[...]
        onehot = (gid == lax.broadcasted_iota(jnp.int32, (1, g), 1)
                  ).astype(jnp.float32)
        psum[...] += jax.lax.dot_general(
            onehot, h, (((0,), (0,)), ((), ())),
            preferred_element_type=jnp.float32)
        cnt[...] += jnp.sum(onehot, axis=0, keepdims=True).T

        @pl.when(i == nblk - 1)
        def _fin():
            gm = psum[...] / jnp.maximum(cnt[...], 1.0)
            gm = jax.nn.relu(jnp.dot(gm, wf1_ref[...],
                                     preferred_element_type=jnp.float32)
                             + bf1_ref[...])
            gm = jax.nn.relu(jnp.dot(gm, wf2_ref[...],
                                     preferred_element_type=jnp.float32)
                             + bf2_ref[...])
            mu = jnp.mean(gm, axis=-1, keepdims=True)
            var = jnp.mean((gm - mu) ** 2, axis=-1, keepdims=True)
            out_ref[...] = ((gm - mu) / jnp.sqrt(var + 1e-5)
                            * gamma_ref[...] + beta_ref[...])

    return pl.pallas_call(
        body,
        grid=(nblk,),
        in_specs=[
            pl.BlockSpec((_BR, f), lambda i: (i, 0)),
            pl.BlockSpec((_BR, f), lambda i: (i, 0)),
            pl.BlockSpec((_BR, 1), lambda i: (i, 0)),
            pl.BlockSpec((1, f), lambda i: (0, 0)),
            pl.BlockSpec((_BR, 1), lambda i: (i, 0)),
            pl.BlockSpec((f, 128), lambda i: (0, 0)),
            pl.BlockSpec((1, 128), lambda i: (0, 0)),
            pl.BlockSpec((128, f), lambda i: (0, 0)),
            pl.BlockSpec((1, f), lambda i: (0, 0)),
            pl.BlockSpec((1, f), lambda i: (0, 0)),
            pl.BlockSpec((1, f), lambda i: (0, 0)),
        ],
        out_specs=pl.BlockSpec((g, f), lambda i: (0, 0)),
        out_shape=jax.ShapeDtypeStruct((g, f), jnp.float32),
        scratch_shapes=[
            pltpu.VMEM((g, f), jnp.float32),
            pltpu.VMEM((g, 1), jnp.float32),
        ],
    )


def kernel(x, edge_index, edge_attr, batch, W1, b1, W2, b2, W3, b3,
           Wf1, bf1, Wf2, bf2, gamma, beta):
    n = x.shape[0]
    e = edge_index.shape[1]
    num_graphs = 256

    e_src = edge_index[0]
    e_dst = edge_index[1]
    deg_parts = _make_deg_kernel(n, e)(e_dst)

    agg64 = _make_agg_kernel(n, e, 64, 25088, 2)
    agg128 = _make_agg_kernel(n, e, 128, 12544, 4)

    dinv = _make_dinv_kernel(n, NC * NS)(deg_parts).reshape(n, 1)
    u1 = _make_t1_kernel(n, 37, 64)(x, dinv, W1)
    agg1 = agg64(u1, e_src, e_dst)
    u2 = _make_t2_kernel(n, 64, 128)(
        agg1[:n], u1, dinv, b1.reshape(1, -1), W2)
    agg2 = agg128(u2, e_src, e_dst)
    u3 = _make_t2_kernel(n, 128, 64)(
        agg2[:n], u2, dinv, b2.reshape(1, -1), W3)
    agg3 = agg64(u3, e_src, e_dst)
    return _make_head_kernel(n, 64, num_graphs)(
        agg3[:n], u3, dinv, b3.reshape(1, -1), batch.reshape(-1, 1),
        Wf1, bf1.reshape(1, -1), Wf2, bf2.reshape(1, -1),
        gamma.reshape(1, -1), beta.reshape(1, -1))
